# baseline (device time: 12042 ns/iter reference)
import jax
import jax.numpy as jnp
from jax import lax
from jax.experimental import pallas as pl
from jax.experimental.pallas import tpu as pltpu

N_DEV = 16
PLANE = 4
COLS = 4


def kernel(x):
    m_per, n = x.shape
    total_rows = N_DEV * m_per

    def body(x_ref, out_ref, plane_acc, col_acc,
             send_sems, recv_p_sems, recv_z_sems, bar_col):
        my = lax.axis_index("i")
        my_p = lax.rem(my, PLANE)
        my_z = my // PLANE

        barrier_sem = pltpu.get_barrier_semaphore()
        for dp in range(1, PLANE):
            tgt = my_z * PLANE + lax.rem(my_p + dp, PLANE)
            pl.semaphore_signal(
                barrier_sem, inc=1,
                device_id=(tgt,), device_id_type=pl.DeviceIdType.MESH,
            )
        for dz in range(1, COLS):
            tgt = lax.rem(my_z + dz, COLS) * PLANE + my_p
            pl.semaphore_signal(
                bar_col, inc=1,
                device_id=(tgt,), device_id_type=pl.DeviceIdType.MESH,
            )

        partial = jnp.sum(x_ref[...].astype(jnp.float32), axis=0, keepdims=True)
        plane_acc[pl.ds(my_p, 1)] = partial.reshape(1, 1, n)

        pl.semaphore_wait(barrier_sem, PLANE - 1)
        sends = []
        for dp in range(1, PLANE):
            tgt = my_z * PLANE + lax.rem(my_p + dp, PLANE)
            rdma = pltpu.make_async_remote_copy(
                src_ref=plane_acc.at[my_p],
                dst_ref=plane_acc.at[my_p],
                send_sem=send_sems.at[dp - 1],
                recv_sem=recv_p_sems.at[my_p],
                device_id=(tgt,),
                device_id_type=pl.DeviceIdType.MESH,
            )
            rdma.start()
            sends.append(rdma)
        for dp in range(1, PLANE):
            src_p = lax.rem(my_p + dp, PLANE)
            recv = pltpu.make_async_remote_copy(
                src_ref=plane_acc.at[my_p],
                dst_ref=plane_acc.at[src_p],
                send_sem=send_sems.at[0],
                recv_sem=recv_p_sems.at[src_p],
                device_id=(my,),
                device_id_type=pl.DeviceIdType.MESH,
            )
            recv.wait_recv()
        plane_sum = jnp.sum(plane_acc[...], axis=0)
        col_acc[pl.ds(my_z, 1)] = plane_sum.reshape(1, 1, n)

        pl.semaphore_wait(bar_col, COLS - 1)
        for dz in range(1, COLS):
            tgt = lax.rem(my_z + dz, COLS) * PLANE + my_p
            rdma = pltpu.make_async_remote_copy(
                src_ref=col_acc.at[my_z],
                dst_ref=col_acc.at[my_z],
                send_sem=send_sems.at[2 + dz],
                recv_sem=recv_z_sems.at[my_z],
                device_id=(tgt,),
                device_id_type=pl.DeviceIdType.MESH,
            )
            rdma.start()
            sends.append(rdma)
        for dz in range(1, COLS):
            src_z = lax.rem(my_z + dz, COLS)
            recv = pltpu.make_async_remote_copy(
                src_ref=col_acc.at[my_z],
                dst_ref=col_acc.at[src_z],
                send_sem=send_sems.at[0],
                recv_sem=recv_z_sems.at[src_z],
                device_id=(my,),
                device_id_type=pl.DeviceIdType.MESH,
            )
            recv.wait_recv()

        out_ref[...] = jnp.sum(col_acc[...], axis=0) * (1.0 / total_rows)

        for rdma in sends:
            rdma.wait_send()

    return pl.pallas_call(
        body,
        out_shape=jax.ShapeDtypeStruct((1, n), jnp.float32),
        in_specs=[pl.BlockSpec(memory_space=pltpu.VMEM)],
        out_specs=pl.BlockSpec(memory_space=pltpu.VMEM),
        scratch_shapes=[
            pltpu.VMEM((PLANE, 1, n), jnp.float32),
            pltpu.VMEM((COLS, 1, n), jnp.float32),
            pltpu.SemaphoreType.DMA((6,)),
            pltpu.SemaphoreType.DMA((PLANE,)),
            pltpu.SemaphoreType.DMA((COLS,)),
            pltpu.SemaphoreType.REGULAR,
        ],
        compiler_params=pltpu.CompilerParams(collective_id=0),
    )(x)


# device time: 11264 ns/iter; 1.0691x vs baseline; 1.0691x over previous
import jax
import jax.numpy as jnp
from jax import lax
from jax.experimental import pallas as pl
from jax.experimental.pallas import tpu as pltpu

N_DEV = 16


def kernel(x):
    m_per, n = x.shape
    total_rows = N_DEV * m_per

    def body(x_ref, out_ref, acc_ref, send_sems, recv_sems):
        my = lax.axis_index("i")

        barrier_sem = pltpu.get_barrier_semaphore()
        for d in range(1, N_DEV):
            tgt = lax.rem(my + d, N_DEV)
            pl.semaphore_signal(
                barrier_sem, inc=1,
                device_id=(tgt,), device_id_type=pl.DeviceIdType.MESH,
            )

        partial = jnp.sum(x_ref[...], axis=0, keepdims=True)
        acc_ref[pl.ds(my, 1)] = partial.reshape(1, 1, n)

        pl.semaphore_wait(barrier_sem, N_DEV - 1)

        sends = []
        for d in range(1, N_DEV):
            tgt = lax.rem(my + d, N_DEV)
            rdma = pltpu.make_async_remote_copy(
                src_ref=acc_ref.at[my],
                dst_ref=acc_ref.at[my],
                send_sem=send_sems.at[d],
                recv_sem=recv_sems.at[my],
                device_id=(tgt,),
                device_id_type=pl.DeviceIdType.MESH,
            )
            rdma.start()
            sends.append(rdma)

        total = partial
        for d in range(1, N_DEV):
            src = lax.rem(my - d + N_DEV, N_DEV)
            recv = pltpu.make_async_remote_copy(
                src_ref=acc_ref.at[my],
                dst_ref=acc_ref.at[src],
                send_sem=send_sems.at[0],
                recv_sem=recv_sems.at[src],
                device_id=(src,),
                device_id_type=pl.DeviceIdType.MESH,
            )
            recv.wait_recv()
            total = total + acc_ref[src]

        out_ref[...] = total * (1.0 / total_rows)

        for rdma in sends:
            rdma.wait_send()

    return pl.pallas_call(
        body,
        out_shape=jax.ShapeDtypeStruct((1, n), jnp.float32),
        in_specs=[pl.BlockSpec(memory_space=pltpu.VMEM)],
        out_specs=pl.BlockSpec(memory_space=pltpu.VMEM),
        scratch_shapes=[
            pltpu.VMEM((N_DEV, 1, n), jnp.float32),
            pltpu.SemaphoreType.DMA((N_DEV,)),
            pltpu.SemaphoreType.DMA((N_DEV,)),
        ],
        compiler_params=pltpu.CompilerParams(collective_id=1),
    )(x)


# device time: 11252 ns/iter; 1.0702x vs baseline; 1.0011x over previous
import jax
import jax.numpy as jnp
from jax import lax
from jax.experimental import pallas as pl
from jax.experimental.pallas import tpu as pltpu

N_DEV = 16


def kernel(x):
    m_per, n = x.shape
    total_rows = N_DEV * m_per

    def body(x_hbm, out_ref, xbuf, copy_sem, acc_ref, send_sems, recv_sems):
        my = lax.axis_index("i")

        barrier_sem = pltpu.get_barrier_semaphore()
        for d in range(1, N_DEV):
            tgt = lax.rem(my + d, N_DEV)
            pl.semaphore_signal(
                barrier_sem, inc=1,
                device_id=(tgt,), device_id_type=pl.DeviceIdType.MESH,
            )

        dma = pltpu.make_async_copy(x_hbm, xbuf, copy_sem)
        dma.start()
        dma.wait()
        partial = jnp.sum(xbuf[...], axis=0, keepdims=True)
        acc_ref[pl.ds(my, 1)] = partial.reshape(1, 1, n)

        pl.semaphore_wait(barrier_sem, N_DEV - 1)

        sends = []
        for d in range(1, N_DEV):
            tgt = lax.rem(my + d, N_DEV)
            rdma = pltpu.make_async_remote_copy(
                src_ref=acc_ref.at[my],
                dst_ref=acc_ref.at[my],
                send_sem=send_sems.at[d],
                recv_sem=recv_sems.at[my],
                device_id=(tgt,),
                device_id_type=pl.DeviceIdType.MESH,
            )
            rdma.start()
            sends.append(rdma)

        total = partial
        for d in range(1, N_DEV):
            src = lax.rem(my - d + N_DEV, N_DEV)
            recv = pltpu.make_async_remote_copy(
                src_ref=acc_ref.at[my],
                dst_ref=acc_ref.at[src],
                send_sem=send_sems.at[0],
                recv_sem=recv_sems.at[src],
                device_id=(src,),
                device_id_type=pl.DeviceIdType.MESH,
            )
            recv.wait_recv()
            total = total + acc_ref[src]

        out_ref[...] = total * (1.0 / total_rows)

        for rdma in sends:
            rdma.wait_send()

    return pl.pallas_call(
        body,
        out_shape=jax.ShapeDtypeStruct((1, n), jnp.float32),
        in_specs=[pl.BlockSpec(memory_space=pl.ANY)],
        out_specs=pl.BlockSpec(memory_space=pltpu.VMEM),
        scratch_shapes=[
            pltpu.VMEM((m_per, n), jnp.float32),
            pltpu.SemaphoreType.DMA,
            pltpu.VMEM((N_DEV, 1, n), jnp.float32),
            pltpu.SemaphoreType.DMA((N_DEV,)),
            pltpu.SemaphoreType.DMA((N_DEV,)),
        ],
        compiler_params=pltpu.CompilerParams(collective_id=1),
    )(x)
